# baseline (device time: 1008071 ns/iter reference)
import jax
import jax.numpy as jnp
from jax import lax
from jax.experimental import pallas as pl
from jax.experimental.pallas import tpu as pltpu

N_DEV = 32
SEQ = 256
D = 1024
HEADS = 8
DH = 128
SCALE = 0.08838834764831843


def kernel(x, Wq, Wo, Wk, Wv):
    def body(x_ref, wq_ref, wo_ref, wk_ref, wv_ref, out_ref,
             msg_ref, send_sems, recv_sems, credit_sems):
        my = lax.axis_index("i")
        left = lax.rem(my - 1 + N_DEV, N_DEV)
        right = lax.rem(my + 1, N_DEV)
        my_dir = lax.rem(my, 2)

        barrier = pltpu.get_barrier_semaphore()
        pl.semaphore_signal(barrier, inc=1, device_id=(left,),
                            device_id_type=pl.DeviceIdType.MESH)
        pl.semaphore_signal(barrier, inc=1, device_id=(right,),
                            device_id_type=pl.DeviceIdType.MESH)
        pl.semaphore_wait(barrier, 2)

        msg_ref[my_dir, 0, 0] = x_ref[0]

        def compute_partial(xv):
            q = jnp.dot(xv, wq_ref[...])
            k = jnp.dot(xv, wk_ref[...])
            v = jnp.dot(xv, wv_ref[...])
            outs = []
            for h in range(HEADS):
                sl = slice(h * DH, (h + 1) * DH)
                s = jnp.dot(q[:, sl], k[:, sl].T) * SCALE
                m = jnp.max(s, axis=-1, keepdims=True)
                p = jnp.exp(s - m)
                p = p / jnp.sum(p, axis=-1, keepdims=True)
                outs.append(jnp.dot(p, v[:, sl]))
            o = jnp.concatenate(outs, axis=1)
            return jnp.dot(o, wo_ref[...])

        def step(t, carry):
            d = lax.rem(my + t, 2)
            sg = lax.rem((t + 1) // 2, 2)
            dsg = lax.rem((t + 2) // 2, 2)
            upstream = jnp.where(d == 0, left, right)
            downstream = jnp.where(d == 0, right, left)

            @pl.when(t >= 2)
            def _():
                prev = pltpu.make_async_remote_copy(
                    src_ref=msg_ref.at[d, 1 - sg],
                    dst_ref=msg_ref.at[d, 1 - sg],
                    send_sem=send_sems.at[d, 1 - sg],
                    recv_sem=recv_sems.at[d, 1 - sg],
                    device_id=(downstream,),
                    device_id_type=pl.DeviceIdType.MESH)
                prev.wait_send()

                @pl.when(t <= N_DEV - 2)
                def _():
                    pl.semaphore_signal(credit_sems.at[d], inc=1,
                                        device_id=(upstream,),
                                        device_id_type=pl.DeviceIdType.MESH)

            recv = pltpu.make_async_remote_copy(
                src_ref=msg_ref.at[d, sg], dst_ref=msg_ref.at[d, sg],
                send_sem=send_sems.at[d, sg], recv_sem=recv_sems.at[d, sg],
                device_id=(upstream,), device_id_type=pl.DeviceIdType.MESH)

            @pl.when(t > 0)
            def _():
                recv.wait_recv()

            partial = compute_partial(msg_ref[d, sg, 0])

            @pl.when(t == 0)
            def _():
                msg_ref[d, sg, 1] = partial

            @pl.when(t > 0)
            def _():
                msg_ref[d, sg, 1] = msg_ref[d, sg, 1] + partial

            @pl.when(t >= 3)
            def _():
                pl.semaphore_wait(credit_sems.at[d], 1)

            send = pltpu.make_async_remote_copy(
                src_ref=msg_ref.at[d, sg], dst_ref=msg_ref.at[d, dsg],
                send_sem=send_sems.at[d, sg], recv_sem=recv_sems.at[d, dsg],
                device_id=(downstream,), device_id_type=pl.DeviceIdType.MESH)
            send.start()
            return carry

        lax.fori_loop(0, N_DEV, step, 0)

        for d_val, s_val in ((my_dir, 1), (1 - my_dir, 0)):
            tail = pltpu.make_async_remote_copy(
                src_ref=msg_ref.at[d_val, s_val],
                dst_ref=msg_ref.at[d_val, s_val],
                send_sem=send_sems.at[d_val, s_val],
                recv_sem=recv_sems.at[d_val, s_val],
                device_id=(right,), device_id_type=pl.DeviceIdType.MESH)
            tail.wait_send()

        final = pltpu.make_async_remote_copy(
            src_ref=msg_ref.at[my_dir, 0], dst_ref=msg_ref.at[my_dir, 0],
            send_sem=send_sems.at[my_dir, 0],
            recv_sem=recv_sems.at[my_dir, 0],
            device_id=(left,), device_id_type=pl.DeviceIdType.MESH)
        final.wait_recv()
        out_ref[0] = msg_ref[my_dir, 0, 1]

    out_shape = jax.ShapeDtypeStruct((1, SEQ, D), jnp.float32)
    return pl.pallas_call(
        body,
        out_shape=out_shape,
        in_specs=[pl.BlockSpec(memory_space=pltpu.VMEM)] * 5,
        out_specs=pl.BlockSpec(memory_space=pltpu.VMEM),
        scratch_shapes=[
            pltpu.VMEM((2, 2, 2, SEQ, D), jnp.float32),
            pltpu.SemaphoreType.DMA((2, 2)),
            pltpu.SemaphoreType.DMA((2, 2)),
            pltpu.SemaphoreType.REGULAR((2,)),
        ],
        compiler_params=pltpu.CompilerParams(collective_id=0),
    )(x, Wq, Wo, Wk, Wv)


# device time: 786839 ns/iter; 1.2812x vs baseline; 1.2812x over previous
import jax
import jax.numpy as jnp
from jax import lax
from jax.experimental import pallas as pl
from jax.experimental.pallas import tpu as pltpu

N_DEV = 32
SEQ = 256
D = 1024
HEADS = 8
DH = 128
SCALE = 0.08838834764831843


def kernel(x, Wq, Wo, Wk, Wv):
    def body(x_ref, wq_ref, wo_ref, wk_ref, wv_ref, out_ref,
             xbuf_ref, accbuf_ref, send_sems_x, send_sems_a,
             recv_sems_x, recv_sems_a, credit_sems):
        my = lax.axis_index("i")
        left = lax.rem(my - 1 + N_DEV, N_DEV)
        right = lax.rem(my + 1, N_DEV)
        my_dir = lax.rem(my, 2)

        barrier = pltpu.get_barrier_semaphore()
        pl.semaphore_signal(barrier, inc=1, device_id=(left,),
                            device_id_type=pl.DeviceIdType.MESH)
        pl.semaphore_signal(barrier, inc=1, device_id=(right,),
                            device_id_type=pl.DeviceIdType.MESH)
        pl.semaphore_wait(barrier, 2)

        def part_copy(d, slot, part, dslot, target):
            buf = xbuf_ref if part == 0 else accbuf_ref
            send_sems = send_sems_x if part == 0 else send_sems_a
            recv_sems = recv_sems_x if part == 0 else recv_sems_a
            return pltpu.make_async_remote_copy(
                src_ref=buf.at[d, slot],
                dst_ref=buf.at[d, dslot],
                send_sem=send_sems.at[d, slot],
                recv_sem=recv_sems.at[d, dslot],
                device_id=(target,), device_id_type=pl.DeviceIdType.MESH)

        def compute_partial(xv):
            q = jnp.dot(xv, wq_ref[...])
            k = jnp.dot(xv, wk_ref[...])
            v = jnp.dot(xv, wv_ref[...])
            outs = []
            for h in range(HEADS):
                sl = slice(h * DH, (h + 1) * DH)
                s = jnp.dot(q[:, sl], k[:, sl].T) * SCALE
                m = jnp.max(s, axis=-1, keepdims=True)
                p = jnp.exp(s - m)
                p = p / jnp.sum(p, axis=-1, keepdims=True)
                outs.append(jnp.dot(p, v[:, sl]))
            o = jnp.concatenate(outs, axis=1)
            return jnp.dot(o, wo_ref[...])

        def step(t, carry):
            d = lax.rem(my + t, 2)
            sg = lax.rem((t + 1) // 2, 2)
            dsg = lax.rem((t + 2) // 2, 2)
            upstream = jnp.where(d == 0, left, right)
            downstream = jnp.where(d == 0, right, left)

            @pl.when(t >= 2)
            def _():
                part_copy(d, 1 - sg, 0, 1 - sg, downstream).wait_send()
                part_copy(d, 1 - sg, 1, 1 - sg, downstream).wait_send()

                @pl.when(t <= N_DEV - 2)
                def _():
                    pl.semaphore_signal(credit_sems.at[d], inc=1,
                                        device_id=(upstream,),
                                        device_id_type=pl.DeviceIdType.MESH)

            @pl.when(t == 0)
            def _():
                xbuf_ref[d, sg] = x_ref[0]

            @pl.when(t > 0)
            def _():
                part_copy(d, sg, 0, sg, upstream).wait_recv()

            @pl.when(t >= 3)
            def _():
                pl.semaphore_wait(credit_sems.at[d], 1)

            part_copy(d, sg, 0, dsg, downstream).start()

            partial = compute_partial(xbuf_ref[d, sg])

            @pl.when(t == 0)
            def _():
                accbuf_ref[d, sg] = partial

            @pl.when(t > 0)
            def _():
                part_copy(d, sg, 1, sg, upstream).wait_recv()
                accbuf_ref[d, sg] = accbuf_ref[d, sg] + partial

            part_copy(d, sg, 1, dsg, downstream).start()
            return carry

        lax.fori_loop(0, N_DEV, step, 0)

        for d_val, s_val in ((my_dir, 1), (1 - my_dir, 0)):
            for part in (0, 1):
                part_copy(d_val, s_val, part, s_val, right).wait_send()

        part_copy(my_dir, 0, 0, 0, left).wait_recv()
        part_copy(my_dir, 0, 1, 0, left).wait_recv()
        out_ref[0] = accbuf_ref[my_dir, 0]

    out_shape = jax.ShapeDtypeStruct((1, SEQ, D), jnp.float32)
    return pl.pallas_call(
        body,
        out_shape=out_shape,
        in_specs=[pl.BlockSpec(memory_space=pltpu.VMEM)] * 5,
        out_specs=pl.BlockSpec(memory_space=pltpu.VMEM),
        scratch_shapes=[
            pltpu.VMEM((2, 2, SEQ, D), jnp.float32),
            pltpu.VMEM((2, 2, SEQ, D), jnp.float32),
            pltpu.SemaphoreType.DMA((2, 2)),
            pltpu.SemaphoreType.DMA((2, 2)),
            pltpu.SemaphoreType.DMA((2, 2)),
            pltpu.SemaphoreType.DMA((2, 2)),
            pltpu.SemaphoreType.REGULAR((2,)),
        ],
        compiler_params=pltpu.CompilerParams(collective_id=0),
    )(x, Wq, Wo, Wk, Wv)


# device time: 446390 ns/iter; 2.2583x vs baseline; 1.7627x over previous
import jax
import jax.numpy as jnp
from jax import lax
from jax.experimental import pallas as pl
from jax.experimental.pallas import tpu as pltpu

N_DEV = 32
SEQ = 256
D = 1024
HEADS = 8
DH = 128
SCALE = 0.08838834764831843

_RING = [0, 3, 4, 7, 15, 12, 11, 8, 16, 19, 20, 23, 31, 28, 27, 24,
         25, 26, 29, 30, 22, 21, 18, 17, 9, 10, 13, 14, 6, 5, 2, 1]
_POS = [0] * N_DEV
_NXT = [0] * N_DEV
_PRV = [0] * N_DEV
for _p, _dev in enumerate(_RING):
    _POS[_dev] = _p
    _NXT[_dev] = _RING[(_p + 1) % N_DEV]
    _PRV[_dev] = _RING[(_p - 1) % N_DEV]


def kernel(x, Wq, Wo, Wk, Wv):
    def body(ring_ref, x_ref, wq_ref, wo_ref, wk_ref, wv_ref, out_ref,
             xbuf_ref, accbuf_ref, send_sems_x, send_sems_a,
             recv_sems_x, recv_sems_a, credit_sems):
        pos = ring_ref[0]
        left = ring_ref[1]
        right = ring_ref[2]
        my_dir = lax.rem(pos, 2)

        barrier = pltpu.get_barrier_semaphore()
        pl.semaphore_signal(barrier, inc=1, device_id=(left,),
                            device_id_type=pl.DeviceIdType.MESH)
        pl.semaphore_signal(barrier, inc=1, device_id=(right,),
                            device_id_type=pl.DeviceIdType.MESH)
        pl.semaphore_wait(barrier, 2)

        def part_copy(d, slot, part, dslot, target):
            buf = xbuf_ref if part == 0 else accbuf_ref
            send_sems = send_sems_x if part == 0 else send_sems_a
            recv_sems = recv_sems_x if part == 0 else recv_sems_a
            return pltpu.make_async_remote_copy(
                src_ref=buf.at[d, slot],
                dst_ref=buf.at[d, dslot],
                send_sem=send_sems.at[d, slot],
                recv_sem=recv_sems.at[d, dslot],
                device_id=(target,), device_id_type=pl.DeviceIdType.MESH)

        def compute_partial(xv):
            q = jnp.dot(xv, wq_ref[...])
            k = jnp.dot(xv, wk_ref[...])
            v = jnp.dot(xv, wv_ref[...])
            outs = []
            for h in range(HEADS):
                sl = slice(h * DH, (h + 1) * DH)
                s = jnp.dot(q[:, sl], k[:, sl].T) * SCALE
                m = jnp.max(s, axis=-1, keepdims=True)
                p = jnp.exp(s - m)
                p = p / jnp.sum(p, axis=-1, keepdims=True)
                outs.append(jnp.dot(p, v[:, sl]))
            o = jnp.concatenate(outs, axis=1)
            return jnp.dot(o, wo_ref[...])

        def step(t, carry):
            d = lax.rem(pos + t, 2)
            sg = lax.rem((t + 1) // 2, 2)
            dsg = lax.rem((t + 2) // 2, 2)
            upstream = jnp.where(d == 0, left, right)
            downstream = jnp.where(d == 0, right, left)

            @pl.when(t >= 2)
            def _():
                part_copy(d, 1 - sg, 0, 1 - sg, downstream).wait_send()
                part_copy(d, 1 - sg, 1, 1 - sg, downstream).wait_send()

                @pl.when(t <= N_DEV - 2)
                def _():
                    pl.semaphore_signal(credit_sems.at[d], inc=1,
                                        device_id=(upstream,),
                                        device_id_type=pl.DeviceIdType.MESH)

            @pl.when(t == 0)
            def _():
                xbuf_ref[d, sg] = x_ref[0]

            @pl.when(t > 0)
            def _():
                part_copy(d, sg, 0, sg, upstream).wait_recv()

            @pl.when(t >= 3)
            def _():
                pl.semaphore_wait(credit_sems.at[d], 1)

            part_copy(d, sg, 0, dsg, downstream).start()

            partial = compute_partial(xbuf_ref[d, sg])

            @pl.when(t == 0)
            def _():
                accbuf_ref[d, sg] = partial

            @pl.when(t > 0)
            def _():
                part_copy(d, sg, 1, sg, upstream).wait_recv()
                accbuf_ref[d, sg] = accbuf_ref[d, sg] + partial

            part_copy(d, sg, 1, dsg, downstream).start()
            return carry

        lax.fori_loop(0, N_DEV, step, 0)

        for d_val, s_val in ((my_dir, 1), (1 - my_dir, 0)):
            for part in (0, 1):
                part_copy(d_val, s_val, part, s_val, right).wait_send()

        part_copy(my_dir, 0, 0, 0, left).wait_recv()
        part_copy(my_dir, 0, 1, 0, left).wait_recv()
        out_ref[0] = accbuf_ref[my_dir, 0]

    my = lax.axis_index("i")
    ring = jnp.stack([
        jnp.take(jnp.array(_POS, jnp.int32), my),
        jnp.take(jnp.array(_PRV, jnp.int32), my),
        jnp.take(jnp.array(_NXT, jnp.int32), my),
    ])

    out_shape = jax.ShapeDtypeStruct((1, SEQ, D), jnp.float32)
    return pl.pallas_call(
        body,
        out_shape=out_shape,
        in_specs=[pl.BlockSpec(memory_space=pltpu.SMEM)]
        + [pl.BlockSpec(memory_space=pltpu.VMEM)] * 5,
        out_specs=pl.BlockSpec(memory_space=pltpu.VMEM),
        scratch_shapes=[
            pltpu.VMEM((2, 2, SEQ, D), jnp.float32),
            pltpu.VMEM((2, 2, SEQ, D), jnp.float32),
            pltpu.SemaphoreType.DMA((2, 2)),
            pltpu.SemaphoreType.DMA((2, 2)),
            pltpu.SemaphoreType.DMA((2, 2)),
            pltpu.SemaphoreType.DMA((2, 2)),
            pltpu.SemaphoreType.REGULAR((2,)),
        ],
        compiler_params=pltpu.CompilerParams(collective_id=0),
    )(ring, x, Wq, Wo, Wk, Wv)


# device time: 439490 ns/iter; 2.2937x vs baseline; 1.0157x over previous
import jax
import jax.numpy as jnp
from jax import lax
from jax.experimental import pallas as pl
from jax.experimental.pallas import tpu as pltpu

N_DEV = 32
SEQ = 256
D = 1024
HEADS = 8
DH = 128
SCALE = 0.08838834764831843

_RING = [0, 3, 4, 7, 15, 12, 11, 8, 16, 19, 20, 23, 31, 28, 27, 24,
         25, 26, 29, 30, 22, 21, 18, 17, 9, 10, 13, 14, 6, 5, 2, 1]
_POS = [0] * N_DEV
_NXT = [0] * N_DEV
_PRV = [0] * N_DEV
for _p, _dev in enumerate(_RING):
    _POS[_dev] = _p
    _NXT[_dev] = _RING[(_p + 1) % N_DEV]
    _PRV[_dev] = _RING[(_p - 1) % N_DEV]


def kernel(x, Wq, Wo, Wk, Wv):
    def body(ring_ref, x_ref, wq_ref, wo_ref, wk_ref, wv_ref, out_ref,
             xbuf_ref, accbuf_ref, send_sems_x, send_sems_a,
             recv_sems_x, recv_sems_a, credit_sems):
        pos = ring_ref[0]
        left = ring_ref[1]
        right = ring_ref[2]
        my_dir = lax.rem(pos, 2)

        barrier = pltpu.get_barrier_semaphore()
        pl.semaphore_signal(barrier, inc=1, device_id=(left,),
                            device_id_type=pl.DeviceIdType.MESH)
        pl.semaphore_signal(barrier, inc=1, device_id=(right,),
                            device_id_type=pl.DeviceIdType.MESH)
        pl.semaphore_wait(barrier, 2)

        def part_copy(d, slot, part, dslot, target):
            buf = xbuf_ref if part == 0 else accbuf_ref
            send_sems = send_sems_x if part == 0 else send_sems_a
            recv_sems = recv_sems_x if part == 0 else recv_sems_a
            return pltpu.make_async_remote_copy(
                src_ref=buf.at[d, slot],
                dst_ref=buf.at[d, dslot],
                send_sem=send_sems.at[d, slot],
                recv_sem=recv_sems.at[d, dslot],
                device_id=(target,), device_id_type=pl.DeviceIdType.MESH)

        def compute_partial(xv):
            q = jnp.dot(xv, wq_ref[...])
            k = jnp.dot(xv, wk_ref[...])
            v = jnp.dot(xv, wv_ref[...])
            outs = []
            for h in range(HEADS):
                sl = slice(h * DH, (h + 1) * DH)
                s = jnp.dot(q[:, sl], k[:, sl].T) * SCALE
                m = jnp.max(s, axis=-1, keepdims=True)
                p = jnp.exp(s - m)
                p = p / jnp.sum(p, axis=-1, keepdims=True)
                outs.append(jnp.dot(p, v[:, sl]))
            o = jnp.concatenate(outs, axis=1)
            return jnp.dot(o, wo_ref[...])

        def step(t, carry):
            d = lax.rem(pos + t, 2)
            sg = lax.rem((t + 1) // 2, 2)
            dsg = lax.rem((t + 2) // 2, 2)
            upstream = jnp.where(d == 0, left, right)
            downstream = jnp.where(d == 0, right, left)

            @pl.when(t >= 2)
            def _():
                part_copy(d, 1 - sg, 0, 1 - sg, downstream).wait_send()
                part_copy(d, 1 - sg, 1, 1 - sg, downstream).wait_send()

                @pl.when(t <= N_DEV - 2)
                def _():
                    pl.semaphore_signal(credit_sems.at[d], inc=1,
                                        device_id=(upstream,),
                                        device_id_type=pl.DeviceIdType.MESH)

            @pl.when(t == 0)
            def _():
                xbuf_ref[d, sg] = x_ref[0].astype(jnp.bfloat16)

            @pl.when(t > 0)
            def _():
                part_copy(d, sg, 0, sg, upstream).wait_recv()

            @pl.when(t >= 3)
            def _():
                pl.semaphore_wait(credit_sems.at[d], 1)

            part_copy(d, sg, 0, dsg, downstream).start()

            partial = compute_partial(xbuf_ref[d, sg].astype(jnp.float32))

            @pl.when(t == 0)
            def _():
                accbuf_ref[d, sg] = partial

            @pl.when(t > 0)
            def _():
                part_copy(d, sg, 1, sg, upstream).wait_recv()
                accbuf_ref[d, sg] = accbuf_ref[d, sg] + partial

            part_copy(d, sg, 1, dsg, downstream).start()
            return carry

        lax.fori_loop(0, N_DEV, step, 0)

        for d_val, s_val in ((my_dir, 1), (1 - my_dir, 0)):
            for part in (0, 1):
                part_copy(d_val, s_val, part, s_val, right).wait_send()

        part_copy(my_dir, 0, 0, 0, left).wait_recv()
        part_copy(my_dir, 0, 1, 0, left).wait_recv()
        out_ref[0] = accbuf_ref[my_dir, 0]

    my = lax.axis_index("i")
    ring = jnp.stack([
        jnp.take(jnp.array(_POS, jnp.int32), my),
        jnp.take(jnp.array(_PRV, jnp.int32), my),
        jnp.take(jnp.array(_NXT, jnp.int32), my),
    ])

    out_shape = jax.ShapeDtypeStruct((1, SEQ, D), jnp.float32)
    return pl.pallas_call(
        body,
        out_shape=out_shape,
        in_specs=[pl.BlockSpec(memory_space=pltpu.SMEM)]
        + [pl.BlockSpec(memory_space=pltpu.VMEM)] * 5,
        out_specs=pl.BlockSpec(memory_space=pltpu.VMEM),
        scratch_shapes=[
            pltpu.VMEM((2, 2, SEQ, D), jnp.bfloat16),
            pltpu.VMEM((2, 2, SEQ, D), jnp.float32),
            pltpu.SemaphoreType.DMA((2, 2)),
            pltpu.SemaphoreType.DMA((2, 2)),
            pltpu.SemaphoreType.DMA((2, 2)),
            pltpu.SemaphoreType.DMA((2, 2)),
            pltpu.SemaphoreType.REGULAR((2,)),
        ],
        compiler_params=pltpu.CompilerParams(collective_id=0),
    )(ring, x, Wq, Wo, Wk, Wv)


# device time: 323648 ns/iter; 3.1147x vs baseline; 1.3579x over previous
import jax
import jax.numpy as jnp
from jax import lax
from jax.experimental import pallas as pl
from jax.experimental.pallas import tpu as pltpu

N_DEV = 32
SEQ = 256
D = 1024
HEADS = 8
DH = 128
SCALE = 0.08838834764831843
NC = 4
CW = D // NC

_RING = [0, 3, 4, 7, 15, 12, 11, 8, 16, 19, 20, 23, 31, 28, 27, 24,
         25, 26, 29, 30, 22, 21, 18, 17, 9, 10, 13, 14, 6, 5, 2, 1]
_POS = [0] * N_DEV
_NXT = [0] * N_DEV
_PRV = [0] * N_DEV
for _p, _dev in enumerate(_RING):
    _POS[_dev] = _p
    _NXT[_dev] = _RING[(_p + 1) % N_DEV]
    _PRV[_dev] = _RING[(_p - 1) % N_DEV]


def kernel(x, Wq, Wo, Wk, Wv):
    def body(ring_ref, x_ref, wq_ref, wo_ref, wk_ref, wv_ref, out_ref,
             xbuf_ref, accbuf_ref, send_sems_x, send_sems_a,
             recv_sems_x, recv_sems_a, credit_sems):
        pos = ring_ref[0]
        left = ring_ref[1]
        right = ring_ref[2]
        my_dir = lax.rem(pos, 2)

        barrier = pltpu.get_barrier_semaphore()
        pl.semaphore_signal(barrier, inc=1, device_id=(left,),
                            device_id_type=pl.DeviceIdType.MESH)
        pl.semaphore_signal(barrier, inc=1, device_id=(right,),
                            device_id_type=pl.DeviceIdType.MESH)
        pl.semaphore_wait(barrier, 2)

        def x_copy(d, slot, dslot, target):
            return pltpu.make_async_remote_copy(
                src_ref=xbuf_ref.at[d, slot],
                dst_ref=xbuf_ref.at[d, dslot],
                send_sem=send_sems_x.at[d, slot],
                recv_sem=recv_sems_x.at[d, dslot],
                device_id=(target,), device_id_type=pl.DeviceIdType.MESH)

        def acc_copy(d, slot, c, dslot, target):
            return pltpu.make_async_remote_copy(
                src_ref=accbuf_ref.at[d, slot * NC + c],
                dst_ref=accbuf_ref.at[d, dslot * NC + c],
                send_sem=send_sems_a.at[d, slot * NC + c],
                recv_sem=recv_sems_a.at[d, dslot * NC + c],
                device_id=(target,), device_id_type=pl.DeviceIdType.MESH)

        def compute_partial(xv):
            q = jnp.dot(xv, wq_ref[...])
            k = jnp.dot(xv, wk_ref[...])
            v = jnp.dot(xv, wv_ref[...])
            outs = []
            for h in range(HEADS):
                sl = slice(h * DH, (h + 1) * DH)
                s = jnp.dot(q[:, sl], k[:, sl].T) * SCALE
                m = jnp.max(s, axis=-1, keepdims=True)
                p = jnp.exp(s - m)
                p = p / jnp.sum(p, axis=-1, keepdims=True)
                outs.append(jnp.dot(p, v[:, sl]))
            o = jnp.concatenate(outs, axis=1)
            return jnp.dot(o, wo_ref[...])

        def step(t, carry):
            d = lax.rem(pos + t, 2)
            sg = lax.rem((t + 1) // 2, 2)
            dsg = lax.rem((t + 2) // 2, 2)
            upstream = jnp.where(d == 0, left, right)
            downstream = jnp.where(d == 0, right, left)

            @pl.when(t >= 2)
            def _():
                x_copy(d, 1 - sg, 1 - sg, downstream).wait_send()
                for c in range(NC):
                    acc_copy(d, 1 - sg, c, 1 - sg, downstream).wait_send()

                @pl.when(t <= N_DEV - 2)
                def _():
                    pl.semaphore_signal(credit_sems.at[d], inc=1,
                                        device_id=(upstream,),
                                        device_id_type=pl.DeviceIdType.MESH)

            @pl.when(t == 0)
            def _():
                xbuf_ref[d, sg] = x_ref[0].astype(jnp.bfloat16)

            @pl.when(t > 0)
            def _():
                x_copy(d, sg, sg, upstream).wait_recv()

            @pl.when(t >= 3)
            def _():
                pl.semaphore_wait(credit_sems.at[d], 1)

            x_copy(d, sg, dsg, downstream).start()

            partial = compute_partial(xbuf_ref[d, sg].astype(jnp.float32))

            for c in range(NC):
                csl = slice(c * CW, (c + 1) * CW)

                @pl.when(t == 0)
                def _():
                    accbuf_ref[d, sg * NC + c] = partial[:, csl]

                @pl.when(t > 0)
                def _():
                    acc_copy(d, sg, c, sg, upstream).wait_recv()
                    accbuf_ref[d, sg * NC + c] = (
                        accbuf_ref[d, sg * NC + c] + partial[:, csl])

                acc_copy(d, sg, c, dsg, downstream).start()
            return carry

        lax.fori_loop(0, N_DEV, step, 0)

        for d_val, s_val in ((my_dir, 1), (1 - my_dir, 0)):
            x_copy(d_val, s_val, s_val, right).wait_send()
            for c in range(NC):
                acc_copy(d_val, s_val, c, s_val, right).wait_send()

        x_copy(my_dir, 0, 0, left).wait_recv()
        for c in range(NC):
            acc_copy(my_dir, 0, c, 0, left).wait_recv()
            out_ref[0, :, c * CW:(c + 1) * CW] = accbuf_ref[my_dir, c]

    my = lax.axis_index("i")
    ring = jnp.stack([
        jnp.take(jnp.array(_POS, jnp.int32), my),
        jnp.take(jnp.array(_PRV, jnp.int32), my),
        jnp.take(jnp.array(_NXT, jnp.int32), my),
    ])

    out_shape = jax.ShapeDtypeStruct((1, SEQ, D), jnp.float32)
    return pl.pallas_call(
        body,
        out_shape=out_shape,
        in_specs=[pl.BlockSpec(memory_space=pltpu.SMEM)]
        + [pl.BlockSpec(memory_space=pltpu.VMEM)] * 5,
        out_specs=pl.BlockSpec(memory_space=pltpu.VMEM),
        scratch_shapes=[
            pltpu.VMEM((2, 2, SEQ, D), jnp.bfloat16),
            pltpu.VMEM((2, 2 * NC, SEQ, CW), jnp.float32),
            pltpu.SemaphoreType.DMA((2, 2)),
            pltpu.SemaphoreType.DMA((2, 2 * NC)),
            pltpu.SemaphoreType.DMA((2, 2)),
            pltpu.SemaphoreType.DMA((2, 2 * NC)),
            pltpu.SemaphoreType.REGULAR((2,)),
        ],
        compiler_params=pltpu.CompilerParams(collective_id=0),
    )(ring, x, Wq, Wo, Wk, Wv)
